# baseline (speedup 1.0000x reference)
import functools
import math

import jax
import jax.numpy as jnp
from jax import lax
from jax.experimental import pallas as pl
from jax.experimental.pallas import tpu as pltpu
from jax.experimental.pallas import tpu_sc as plsc

_D = 64
_IDX_W = 128

_info = plsc.get_sparse_core_info()
_NC, _NS = _info.num_cores, _info.num_subcores
_NW = _NC * _NS


def _sc_gather_sum(nrows: int):
    rows_per_w = nrows // _NW
    nhalves = 2
    rows_per_h = rows_per_w // nhalves
    half = rows_per_h // 2
    mesh = plsc.VectorSubcoreMesh(core_axis_name="c", subcore_axis_name="s")

    @functools.partial(
        pl.kernel,
        mesh=mesh,
        compiler_params=pltpu.CompilerParams(use_tc_tiling_on_sc=False),
        out_type=jax.ShapeDtypeStruct((nrows * _IDX_W // 2, 128), jnp.float32),
        scratch_types=[
            pltpu.VMEM((rows_per_h, _IDX_W), jnp.int32),
            pltpu.VMEM((rows_per_h, _IDX_W), jnp.int32),
            pltpu.VMEM((rows_per_h, _IDX_W), jnp.int32),
            pltpu.VMEM((2, _IDX_W, _D), jnp.float32),
            pltpu.VMEM((2, _IDX_W, _D), jnp.float32),
            pltpu.VMEM((2, _IDX_W, _D), jnp.float32),
            pltpu.VMEM((2, _IDX_W // 2, 128), jnp.float32),
            pltpu.SemaphoreType.DMA,
            pltpu.SemaphoreType.DMA,
            pltpu.SemaphoreType.DMA,
            pltpu.SemaphoreType.DMA,
            pltpu.SemaphoreType.DMA,
            pltpu.SemaphoreType.DMA,
            pltpu.SemaphoreType.DMA,
            pltpu.SemaphoreType.DMA,
        ],
    )
    def k(t0, t1, t2, e0, e1, e2, out,
          I0, I1, I2, r0, r1, r2, p,
          g00, g01, g02, g10, g11, g12, w0, w1):
        wid = lax.axis_index("s") * _NC + lax.axis_index("c")
        row0 = wid * rows_per_w
        gsems = ((g00, g01, g02), (g10, g11, g12))
        wsems = (w0, w1)

        def run_half(hbase):
            def issue(c, s):
                pltpu.async_copy(e0.at[I0.at[c]], r0.at[s], gsems[s][0])
                pltpu.async_copy(e1.at[I1.at[c]], r1.at[s], gsems[s][1])
                pltpu.async_copy(e2.at[I2.at[c]], r2.at[s], gsems[s][2])

            def drain_w(s):
                pltpu.make_async_copy(
                    out.at[pl.ds(0, _IDX_W // 2), :], p.at[s], wsems[s]).wait()

            def complete(c, s):
                for m in range(3):
                    pltpu.make_async_copy(
                        e0.at[pl.ds(0, _IDX_W)], r0.at[s], gsems[s][m]).wait()

                def add_pair(jj, carry):
                    for half_tok in range(2):
                        i = 2 * jj + half_tok
                        for j in range(_D // 16):
                            sl = pl.ds(j * 16, 16)
                            dl = pl.ds(half_tok * _D + j * 16, 16)
                            p[s, jj, dl] = (
                                r0[s, i, sl] + r1[s, i, sl] + r2[s, i, sl]
                            )
                    return carry

                lax.fori_loop(0, _IDX_W // 2, add_pair, 0)
                pltpu.async_copy(
                    p.at[s],
                    out.at[pl.ds((hbase + c) * (_IDX_W // 2), _IDX_W // 2), :],
                    wsems[s])

            pltpu.sync_copy(t0.at[pl.ds(hbase, rows_per_h), :], I0)
            pltpu.sync_copy(t1.at[pl.ds(hbase, rows_per_h), :], I1)
            pltpu.sync_copy(t2.at[pl.ds(hbase, rows_per_h), :], I2)

            issue(0, 0)

            def body(g, carry):
                c1 = 2 * g + 1

                @pl.when(g >= 1)
                def _():
                    drain_w(1)

                issue(c1, 1)
                complete(2 * g, 0)
                c2 = 2 * g + 2

                @pl.when(c2 < rows_per_h)
                def _():
                    drain_w(0)
                    issue(c2, 0)

                complete(c1, 1)
                return carry

            lax.fori_loop(0, half, body, 0)
            drain_w(0)
            drain_w(1)

        for ih in range(nhalves):
            run_half(row0 + ih * rows_per_h)

    return k


def _tc_mlp_add(n2: int, bn2: int):
    grid = n2 // bn2
    inv_sqrt2 = 1.0 / math.sqrt(2.0)

    def body(h_ref, x_ref, w1_ref, b1_ref, w2_ref, b2_ref, o_ref):
        x = x_ref[...]
        y = jnp.dot(x, w1_ref[...], preferred_element_type=jnp.float32)
        y = y + b1_ref[...]
        y = 0.5 * y * (1.0 + lax.erf(y * inv_sqrt2))
        z = jnp.dot(y, w2_ref[...], preferred_element_type=jnp.float32)
        o_ref[...] = h_ref[...] + z + b2_ref[...]

    return pl.pallas_call(
        body,
        grid=(grid,),
        in_specs=[
            pl.BlockSpec((bn2, 128), lambda i: (i, 0)),
            pl.BlockSpec((bn2, 128), lambda i: (i, 0)),
            pl.BlockSpec((128, 128), lambda i: (0, 0)),
            pl.BlockSpec((1, 128), lambda i: (0, 0)),
            pl.BlockSpec((128, 128), lambda i: (0, 0)),
            pl.BlockSpec((1, 128), lambda i: (0, 0)),
        ],
        out_specs=pl.BlockSpec((bn2, 128), lambda i: (i, 0)),
        out_shape=jax.ShapeDtypeStruct((n2, 128), jnp.float32),
        compiler_params=pltpu.CompilerParams(
            dimension_semantics=("arbitrary",),
        ),
    )


def _blockdiag2(w):
    z = jnp.zeros_like(w)
    return jnp.concatenate(
        [jnp.concatenate([w, z], axis=1), jnp.concatenate([z, w], axis=1)],
        axis=0,
    )


def kernel(tokens_l0, tokens_l1, tokens_l2, time_embed, E0, E1, E2, W1, b1, W2, b2):
    b, l, d = time_embed.shape
    n = b * l
    nrows = n // _IDX_W

    t0 = tokens_l0.reshape(nrows, _IDX_W).astype(jnp.int32)
    t1 = tokens_l1.reshape(nrows, _IDX_W).astype(jnp.int32)
    t2 = tokens_l2.reshape(nrows, _IDX_W).astype(jnp.int32)

    h = _sc_gather_sum(nrows)(t0, t1, t2, E0, E1, E2)

    x_p = time_embed.reshape(n // 2, 2 * d)
    w1b = _blockdiag2(W1.T)
    w2b = _blockdiag2(W2.T)
    b1b = jnp.concatenate([b1, b1]).reshape(1, 2 * d)
    b2b = jnp.concatenate([b2, b2]).reshape(1, 2 * d)

    out = _tc_mlp_add(n // 2, 12800)(h, x_p, w1b, b1b, w2b, b2b)
    return out.reshape(b, l, d)
